# Initial kernel scaffold; baseline (speedup 1.0000x reference)
import jax
import jax.numpy as jnp
from jax.experimental import pallas as pl


def kernel(x, topk_ids, topk_weight, gate_w, up_w, down_w):
    raise NotImplementedError("write your pallas kernel here")


# speedup vs baseline: 1.3997x; 1.3997x over previous
import jax
import jax.numpy as jnp
from jax import lax
from jax.experimental import pallas as pl
from jax.experimental.pallas import tpu as pltpu
from jax.experimental.pallas import tpu_sc as plsc

TM = 128
TF = 256
CH = 16
SC_NC = 2
SC_NS = 16
PREC = jax.lax.Precision.HIGHEST


def _sc_gather_rows(x, idx):
    n, d = x.shape
    nw = SC_NC * SC_NS
    b_per_w = n // nw
    mesh = plsc.VectorSubcoreMesh(core_axis_name="c", subcore_axis_name="s")

    @pl.kernel(
        out_type=jax.ShapeDtypeStruct((n, d), x.dtype),
        mesh=mesh,
        scratch_types=[
            pltpu.VMEM((CH,), jnp.int32),
            pltpu.VMEM((CH, d), x.dtype),
            pltpu.SemaphoreType.DMA,
        ],
    )
    def k(x_hbm, i_hbm, o_hbm, idx_v, rows_v, sem):
        wid = lax.axis_index("s") * SC_NC + lax.axis_index("c")
        base = wid * b_per_w

        @pl.loop(0, b_per_w, step=CH)
        def _(c):
            pltpu.sync_copy(i_hbm.at[pl.ds(base + c, CH)], idx_v)
            pltpu.async_copy(x_hbm.at[idx_v], rows_v, sem).wait()
            pltpu.sync_copy(rows_v, o_hbm.at[pl.ds(base + c, CH)])

    return k(x, idx)


def _sc_scatter_rows(ys, idx):
    n, d = ys.shape
    nw = SC_NC * SC_NS
    b_per_w = n // nw
    mesh = plsc.VectorSubcoreMesh(core_axis_name="c", subcore_axis_name="s")

    @pl.kernel(
        out_type=jax.ShapeDtypeStruct((n, d), ys.dtype),
        mesh=mesh,
        scratch_types=[
            pltpu.VMEM((CH,), jnp.int32),
            pltpu.VMEM((CH, d), ys.dtype),
            pltpu.SemaphoreType.DMA,
        ],
    )
    def k(y_hbm, i_hbm, o_hbm, idx_v, rows_v, sem):
        wid = lax.axis_index("s") * SC_NC + lax.axis_index("c")
        base = wid * b_per_w

        @pl.loop(0, b_per_w, step=CH)
        def _(c):
            pltpu.sync_copy(i_hbm.at[pl.ds(base + c, CH)], idx_v)
            pltpu.sync_copy(y_hbm.at[pl.ds(base + c, CH)], rows_v)
            pltpu.async_copy(rows_v, o_hbm.at[idx_v], sem).wait()

    return k(ys, idx)


def _ffn_kernel(ts_ref, tc_ref, eid_ref, rw_ref, xs_ref, gw_ref, uw_ref, dw_ref,
                ys_ref):
    e = pl.program_id(0)
    f = pl.program_id(1)

    @pl.when(jnp.logical_and(e == 0, f == 0))
    def _init():
        ys_ref[...] = jnp.zeros(ys_ref.shape, ys_ref.dtype)

    t0 = ts_ref[e]
    cnt = tc_ref[e]
    gw = gw_ref[0]
    uw = uw_ref[0]
    dw = dw_ref[0]

    def body(i, carry):
        r0 = (t0 + i) * TM
        xb = xs_ref[pl.ds(r0, TM), :]
        g = jax.lax.dot_general(xb, gw, (((1,), (1,)), ((), ())),
                                preferred_element_type=jnp.float32,
                                precision=PREC)
        u = jax.lax.dot_general(xb, uw, (((1,), (1,)), ((), ())),
                                preferred_element_type=jnp.float32,
                                precision=PREC)
        a = (g * jax.lax.logistic(g)) * u
        h = jax.lax.dot_general(a, dw, (((1,), (1,)), ((), ())),
                                preferred_element_type=jnp.float32,
                                precision=PREC)
        sel = eid_ref[pl.ds(r0, TM), :] == e
        scale = jnp.where(sel, rw_ref[pl.ds(r0, TM), :], 0.0)
        ys_ref[pl.ds(r0, TM), :] += h * scale
        return carry

    jax.lax.fori_loop(0, cnt, body, 0)


def _grouped_ffn(xs, eid_s, rw_s, gate_w, up_w, down_w, tile_start, tile_cnt):
    n, d = xs.shape
    e, f_dim, _ = gate_w.shape
    grid = (e, f_dim // TF)
    spec = pltpu.PrefetchScalarGridSpec(
        num_scalar_prefetch=2,
        grid=grid,
        in_specs=[
            pl.BlockSpec((n, 1), lambda i, j, ts, tc: (0, 0)),
            pl.BlockSpec((n, 1), lambda i, j, ts, tc: (0, 0)),
            pl.BlockSpec((n, d), lambda i, j, ts, tc: (0, 0)),
            pl.BlockSpec((1, TF, d), lambda i, j, ts, tc: (i, j, 0)),
            pl.BlockSpec((1, TF, d), lambda i, j, ts, tc: (i, j, 0)),
            pl.BlockSpec((1, d, TF), lambda i, j, ts, tc: (i, 0, j)),
        ],
        out_specs=pl.BlockSpec((n, d), lambda i, j, ts, tc: (0, 0)),
    )
    return pl.pallas_call(
        _ffn_kernel,
        grid_spec=spec,
        out_shape=jax.ShapeDtypeStruct((n, d), jnp.float32),
        compiler_params=pltpu.CompilerParams(
            dimension_semantics=("arbitrary", "arbitrary"),
        ),
    )(tile_start, tile_cnt, eid_s.reshape(n, 1), rw_s.reshape(n, 1), xs,
      gate_w, up_w, down_w)


def kernel(x, topk_ids, topk_weight, gate_w, up_w, down_w):
    n, _ = x.shape
    e = gate_w.shape[0]

    eid = topk_ids[:, 0].astype(jnp.int32)
    sort_idx = jnp.argsort(eid).astype(jnp.int32)
    eid_s = jnp.sort(eid)
    rw_s = topk_weight[sort_idx, 0]
    erange = jnp.arange(e, dtype=jnp.int32)
    starts = jnp.searchsorted(eid_s, erange, side="left").astype(jnp.int32)
    ends = jnp.searchsorted(eid_s, erange, side="right").astype(jnp.int32)
    tile_start = (starts // TM).astype(jnp.int32)
    tile_cnt = jnp.where(ends > starts, (ends - 1) // TM - starts // TM + 1,
                         0).astype(jnp.int32)

    xs = _sc_gather_rows(x, sort_idx)
    ys = _grouped_ffn(xs, eid_s, rw_s, gate_w, up_w, down_w, tile_start,
                      tile_cnt)
    return _sc_scatter_rows(ys, sort_idx)
